# baseline (speedup 1.0000x reference)
import functools

import jax
import jax.numpy as jnp
from jax import lax
from jax.experimental import pallas as pl
from jax.experimental.pallas import tpu as pltpu
from jax.experimental.pallas import tpu_sc as plsc

B = 16384
F = 26
FH = F // 2
NC = 2
NS = 16
NW = NC * NS
BPW = B // NW
NH = FH * BPW

_mesh = plsc.VectorSubcoreMesh(core_axis_name="c", subcore_axis_name="s")


@functools.partial(
    pl.kernel,
    out_type=jax.ShapeDtypeStruct((B,), jnp.float32),
    mesh=_mesh,
    scratch_types=[
        *[pltpu.VMEM((2 * BPW,), jnp.int32) for _ in range(13)],
        pltpu.VMEM((F, BPW), jnp.float32),
        *[pltpu.VMEM((2 * BPW,), jnp.float32) for _ in range(13)],
        pltpu.VMEM((BPW,), jnp.float32),
        pltpu.SemaphoreType.DMA,
        *[pltpu.SemaphoreType.DMA for _ in range(13)],
    ],
)
def _sc_kernel(idx_hbm, vals_hbm, table_hbm, out_hbm, *scr):
    idx_ck = scr[0:13]
    vals_v = scr[13]
    g_ck = scr[14:27]
    acc_v = scr[27]
    sem_s = scr[28]
    sems = scr[29:42]
    wid = lax.axis_index("s") * NC + lax.axis_index("c")
    base = wid * BPW

    stage = [
        pltpu.async_copy(
            idx_hbm.at[f, pl.ds(base, BPW)],
            idx_ck[f // 2].at[pl.ds((f % 2) * BPW, BPW)],
            sem_s,
        )
        for f in range(F)
    ]
    gathers = []
    for k in range(13):
        stage[2 * k].wait()
        stage[2 * k + 1].wait()
        gathers.append(
            pltpu.async_copy(table_hbm.at[idx_ck[k]], g_ck[k], sems[k])
        )
    pltpu.sync_copy(vals_hbm.at[:, pl.ds(base, BPW)], vals_v)

    def _chunk(g_v, f0, first):
        def _body(i, carry):
            sl16 = pl.ds(i * 16, 16)
            acc = jnp.zeros((16,), jnp.float32) if first else acc_v[sl16]
            for fh in range(2):
                acc = acc + (g_v[pl.ds(fh * BPW + i * 16, 16)]
                             * vals_v[f0 + fh, sl16])
            acc_v[sl16] = acc
            return carry

        lax.fori_loop(0, BPW // 16, _body, 0)

    for k in range(13):
        gathers[k].wait()
        _chunk(g_ck[k], 2 * k, k == 0)

    pltpu.sync_copy(acc_v, out_hbm.at[pl.ds(base, BPW)])


def kernel(feature_idx, feature_values, table):
    idx_t = feature_idx.astype(jnp.int32).T
    vals_t = feature_values.T
    return _sc_kernel(idx_t, vals_t, table.reshape(-1))


# speedup vs baseline: 1.0238x; 1.0238x over previous
import functools

import jax
import jax.numpy as jnp
from jax import lax
from jax.experimental import pallas as pl
from jax.experimental.pallas import tpu as pltpu
from jax.experimental.pallas import tpu_sc as plsc

B = 16384
F = 26
FH = F // 2
NC = 2
NS = 16
NW = NC * NS
BPW = B // NW
NH = FH * BPW

_mesh = plsc.VectorSubcoreMesh(core_axis_name="c", subcore_axis_name="s")


@functools.partial(
    pl.kernel,
    out_type=jax.ShapeDtypeStruct((B,), jnp.float32),
    mesh=_mesh,
    scratch_types=[
        pltpu.VMEM((NH,), jnp.int32),
        pltpu.VMEM((NH,), jnp.int32),
        pltpu.VMEM((F, BPW), jnp.float32),
        pltpu.VMEM((NH,), jnp.float32),
        pltpu.VMEM((NH,), jnp.float32),
        pltpu.VMEM((BPW,), jnp.float32),
        pltpu.SemaphoreType.DMA,
        pltpu.SemaphoreType.DMA,
        pltpu.SemaphoreType.DMA,
    ],
)
def _sc_kernel(idx_hbm, vals_hbm, table_hbm, out_hbm,
               idxa_v, idxb_v, vals_v, ga_v, gb_v, acc_v,
               sem_s, sem_a, sem_b):
    wid = lax.axis_index("s") * NC + lax.axis_index("c")
    base = wid * BPW

    stage = [
        pltpu.async_copy(
            idx_hbm.at[f, pl.ds(base, BPW)],
            (idxa_v if f < FH else idxb_v).at[pl.ds((f % FH) * BPW, BPW)],
            sem_s,
        )
        for f in range(F)
    ]
    for cp in stage[:FH]:
        cp.wait()
    cpa = pltpu.async_copy(table_hbm.at[idxa_v], ga_v, sem_a)
    for cp in stage[FH:]:
        cp.wait()
    cpb = pltpu.async_copy(table_hbm.at[idxb_v], gb_v, sem_b)
    pltpu.sync_copy(vals_hbm.at[:, pl.ds(base, BPW)], vals_v)

    def _half(g_v, f0, first):
        def _body(i, carry):
            sl16 = pl.ds(i * 16, 16)
            acc = jnp.zeros((16,), jnp.float32) if first else acc_v[sl16]
            for fh in range(FH):
                acc = acc + (g_v[pl.ds(fh * BPW + i * 16, 16)]
                             * vals_v[f0 + fh, sl16])
            acc_v[sl16] = acc
            return carry

        lax.fori_loop(0, BPW // 16, _body, 0)

    cpa.wait()
    _half(ga_v, 0, True)
    cpb.wait()
    _half(gb_v, FH, False)

    pltpu.sync_copy(acc_v, out_hbm.at[pl.ds(base, BPW)])


def kernel(feature_idx, feature_values, table):
    idx_t = feature_idx.astype(jnp.int32).T
    vals_t = feature_values.T
    return _sc_kernel(idx_t, vals_t, table.reshape(-1))
